# baseline (speedup 1.0000x reference)
import jax
import jax.numpy as jnp
import numpy as np
from jax.experimental import pallas as pl

VOXEL_SIZE = np.array([0.05, 0.05, 0.1], dtype=np.float32)
PC_RANGE = np.array([0.0, -40.0, -3.0, 70.4, 40.0, 1.0], dtype=np.float32)
MAX_VOXELS = 20000
MAX_OCCUPANCY = 5
GRID_SIZE = np.round((PC_RANGE[3:] - PC_RANGE[:3]) / VOXEL_SIZE).astype(np.int32)
GX, GY, GZ = int(GRID_SIZE[0]), int(GRID_SIZE[1]), int(GRID_SIZE[2])
BIG = GX * GY * GZ


def _vid_kernel(p_ref, vid_ref):
    p = p_ref[0]
    cx = jnp.floor((p[0, :] - float(PC_RANGE[0])) / float(VOXEL_SIZE[0])).astype(jnp.int32)
    cy = jnp.floor((p[1, :] - float(PC_RANGE[1])) / float(VOXEL_SIZE[1])).astype(jnp.int32)
    cz = jnp.floor((p[2, :] - float(PC_RANGE[2])) / float(VOXEL_SIZE[2])).astype(jnp.int32)
    valid = ((cx >= 0) & (cx < GX) & (cy >= 0) & (cy < GY)
             & (cz >= 0) & (cz < GZ))
    vid = (cz * GY + cy) * GX + cx
    vid_ref[0, 0, :] = jnp.where(valid, vid, BIG)


def _compute_vids(points):
    B, Np, _ = points.shape
    pt = jnp.transpose(points, (0, 2, 1))
    out = pl.pallas_call(
        _vid_kernel,
        grid=(B,),
        in_specs=[pl.BlockSpec((1, 4, Np), lambda i: (i, 0, 0))],
        out_specs=pl.BlockSpec((1, 1, Np), lambda i: (i, 0, 0)),
        out_shape=jax.ShapeDtypeStruct((B, 1, Np), jnp.int32),
    )(pt)
    return out.reshape(B, Np)


def _voxelize_one(p, vid):
    Np = p.shape[0]
    order = jnp.argsort(vid)
    svid = vid[order]
    sp = p[order]
    svalid = svid < BIG
    is_new = jnp.concatenate([jnp.array([True]), svid[1:] != svid[:-1]]) & svalid
    slot = jnp.cumsum(is_new.astype(jnp.int32)) - 1
    idx = jnp.arange(Np, dtype=jnp.int32)
    seg_start = jax.lax.cummax(jnp.where(is_new, idx, -1), axis=0)
    rank = idx - seg_start
    keep = svalid & (slot >= 0) & (slot < MAX_VOXELS) & (rank < MAX_OCCUPANCY)
    slot_w = jnp.where(keep, slot, MAX_VOXELS)
    rank_w = jnp.where(keep, rank, MAX_OCCUPANCY)
    features = jnp.zeros((MAX_VOXELS, MAX_OCCUPANCY, p.shape[1]), dtype=p.dtype)
    features = features.at[slot_w, rank_w].set(sp, mode='drop')
    occupancy = jnp.zeros((MAX_VOXELS,), dtype=jnp.int32).at[slot_w].add(
        keep.astype(jnp.int32), mode='drop')
    cz = svid // (GX * GY)
    cy = (svid // GX) % GY
    cx = svid % GX
    coords_pt = jnp.stack([cz, cy, cx], axis=1)
    first_slot = jnp.where(is_new & (slot < MAX_VOXELS), slot, MAX_VOXELS)
    coordinates = jnp.zeros((MAX_VOXELS, 3), dtype=jnp.int32).at[first_slot].set(
        coords_pt.astype(jnp.int32), mode='drop')
    return features, coordinates, occupancy


def kernel(points):
    B = points.shape[0]
    vids = _compute_vids(points)
    feats, coords, occs = [], [], []
    for i in range(B):
        f, c, o = _voxelize_one(points[i], vids[i])
        c = jnp.pad(c, ((0, 0), (1, 0)), constant_values=i)
        feats.append(f)
        coords.append(c)
        occs.append(o)
    features = jnp.concatenate(feats, axis=0)
    coordinates = jnp.concatenate(coords, axis=0)
    occupancy = jnp.concatenate(occs, axis=0)
    return points, features, coordinates, occupancy
